# Initial kernel scaffold; baseline (speedup 1.0000x reference)
import jax
import jax.numpy as jnp
from jax.experimental import pallas as pl


def kernel(atom_features, pair_features, pair_split, atom_to_pair, num_atoms, W_pap, b_pap, W_pa, b_pa, W_aa, b_aa, W_ao, b_ao, W_ap, b_ap, W_pp, b_pp, W_po, b_po):
    raise NotImplementedError("write your pallas kernel here")


# speedup vs baseline: 1.8771x; 1.8771x over previous
import functools

import jax
import jax.numpy as jnp
from jax import lax
from jax.experimental import pallas as pl
from jax.experimental.pallas import tpu as pltpu
from jax.experimental.pallas import tpu_sc as plsc

_ALPHA = 0.1

_NC = 2
_NS = 16
_NW = _NC * _NS
_K = 128
_ROW = 128


def _leaky(x):
    return jnp.where(x > 0, x, _ALPHA * x)


def _tc_pre(atom_features, pair_features, W_cat, W_pap_p, b_pap, pair_block):
    n_atoms, d_atom = atom_features.shape
    n_pairs, d_pair = pair_features.shape
    d_agg = W_pap_p.shape[1]
    d_cat = W_cat.shape[1]
    grid = n_pairs // pair_block

    def body(pf_ref, af_ref, wcat_ref, wpp_ref, bpap_ref, tp_ref, t_ref):
        pid = pl.program_id(0)
        tp_ref[...] = (
            jnp.dot(pf_ref[...], wpp_ref[...],
                    preferred_element_type=jnp.float32) + bpap_ref[...])

        @pl.when(pid == 0)
        def _():
            t_ref[...] = jnp.dot(af_ref[...], wcat_ref[...],
                                 preferred_element_type=jnp.float32)

    return pl.pallas_call(
        body,
        grid=(grid,),
        in_specs=[
            pl.BlockSpec((pair_block, d_pair), lambda i: (i, 0)),
            pl.BlockSpec((n_atoms, d_atom), lambda i: (0, 0)),
            pl.BlockSpec((d_atom, d_cat), lambda i: (0, 0)),
            pl.BlockSpec((d_pair, d_agg), lambda i: (0, 0)),
            pl.BlockSpec((1, d_agg), lambda i: (0, 0)),
        ],
        out_specs=[
            pl.BlockSpec((pair_block, d_agg), lambda i: (i, 0)),
            pl.BlockSpec((n_atoms, d_cat), lambda i: (0, 0)),
        ],
        out_shape=[
            jax.ShapeDtypeStruct((n_pairs, d_agg), jnp.float32),
            jax.ShapeDtypeStruct((n_atoms, d_cat), jnp.float32),
        ],
    )(pair_features, atom_features, W_cat, W_pap_p, b_pap)


def _sc_gather_scatter(pair_i, pair_j, T, Tp, d_agg, d_out):
    n_pairs = pair_i.shape[0]
    n_atoms, d_cat = T.shape
    assert n_pairs % _K == 0
    n_chunks = n_pairs // _K
    H = -(-n_atoms // (_NC * _NS * 8)) * (_NS * 8)
    rows_per_sub = H // _NS
    acc_rows = H + 8
    half_chunks = n_chunks // 2

    mesh = plsc.VectorSubcoreMesh(core_axis_name="c", subcore_axis_name="s")

    @functools.partial(
        pl.kernel,
        out_type=[
            jax.ShapeDtypeStruct((_NC * H, _ROW), jnp.float32),
            jax.ShapeDtypeStruct((n_pairs, d_out), jnp.float32),
        ],
        mesh=mesh,
        scratch_types=[
            pltpu.VMEM((_K,), jnp.int32),
            pltpu.VMEM((_K,), jnp.int32),
            pltpu.VMEM((_K,), jnp.int32),
            pltpu.VMEM((_K, d_cat), jnp.float32),
            pltpu.VMEM((_K, d_cat), jnp.float32),
            pltpu.VMEM((_K, d_agg), jnp.float32),
            pltpu.VMEM((_K, _ROW), jnp.float32),
            pltpu.VMEM((_K, d_out), jnp.float32),
            pltpu.VMEM_SHARED((acc_rows, _ROW), jnp.float32),
            pltpu.SemaphoreType.DMA,
            pltpu.SemaphoreType.DMA,
            pltpu.SemaphoreType.DMA,
        ],
    )
    def sc_kernel(pi_hbm, pj_hbm, t_hbm, tp_hbm,
                  s_out, vv_out,
                  idx_i, idx_j, idx_s, buf_ti, buf_tj, buf_tp, buf_a,
                  buf_vv, s_sh,
                  sem0, sem1, sem2):
        cid = lax.axis_index("c")
        sid = lax.axis_index("s")

        zero16 = jnp.zeros((16,), jnp.float32)
        row_lo = cid * H

        def zrow(r, carry):
            for cc in range(_ROW // 16):
                buf_a[r, pl.ds(cc * 16, 16)] = zero16
            return carry

        lax.fori_loop(0, _K, zrow, 0)
        z0 = sid * rows_per_sub
        off = 0
        for sz in (_K, _K, rows_per_sub + 8 - 2 * _K):
            pltpu.sync_copy(buf_a.at[pl.ds(0, sz)],
                            s_sh.at[pl.ds(z0 + off, sz)])
            off += sz
        plsc.subcore_barrier()

        def chunk_body(t, carry):
            c = t * _NS + sid
            base = c * _K
            pltpu.sync_copy(pi_hbm.at[pl.ds(base, _K)], idx_i)
            pltpu.sync_copy(pj_hbm.at[pl.ds(base, _K)], idx_j)
            cp_tp = pltpu.async_copy(tp_hbm.at[pl.ds(base, _K)], buf_tp, sem0)
            g_ti = pltpu.async_copy(t_hbm.at[idx_i], buf_ti, sem1)
            g_tj = pltpu.async_copy(t_hbm.at[idx_j], buf_tj, sem2)

            def irow(k, c2):
                sl = pl.ds(k * 16, 16)
                v = idx_i[sl] - row_lo
                ok = (v >= 0) & (v < H)
                idx_s[sl] = jnp.where(ok, v, H)
                return c2

            lax.fori_loop(0, _K // 16, irow, 0)

            cp_tp.wait()
            g_ti.wait()
            g_tj.wait()

            def arow(r, c2):
                for cc in range(d_agg // 16):
                    a = (buf_ti[r, pl.ds(cc * 16, 16)]
                         + buf_tj[r, pl.ds(d_agg + cc * 16, 16)]
                         + buf_tp[r, pl.ds(cc * 16, 16)])
                    buf_a[r, pl.ds(cc * 16, 16)] = jnp.where(
                        a > 0, a, _ALPHA * a)
                return c2

            lax.fori_loop(0, _K, arow, 0)
            pltpu.sync_copy(buf_a, s_sh.at[idx_s], add=True)

            @pl.when((c < half_chunks) == (cid == 0))
            def _():
                def vrow(r, c2):
                    for cc in range(d_out // 16):
                        sl = pl.ds(2 * d_agg + cc * 16, 16)
                        buf_vv[r, pl.ds(cc * 16, 16)] = (
                            buf_ti[r, sl] + buf_tj[r, sl])
                    return c2

                lax.fori_loop(0, _K, vrow, 0)
                pltpu.sync_copy(buf_vv, vv_out.at[pl.ds(base, _K)])

            return carry

        n_rem = n_chunks % _NS
        n_mine = n_chunks // _NS + jnp.where(sid < n_rem, 1, 0)
        lax.fori_loop(0, n_mine, chunk_body, 0)

        plsc.subcore_barrier()
        r0 = sid * rows_per_sub
        off = 0
        for sz in (_K, _K, rows_per_sub - 2 * _K):
            pltpu.sync_copy(s_sh.at[pl.ds(r0 + off, sz)],
                            buf_a.at[pl.ds(0, sz)])
            pltpu.sync_copy(buf_a.at[pl.ds(0, sz)],
                            s_out.at[pl.ds(cid * H + r0 + off, sz)])
            off += sz

    return sc_kernel(pair_i, pair_j, T, Tp)


def _tc_atom(atom_features, s_part, W_pa_a, W_pa_s, b_pa, W_aa, b_aa,
             W_ao1, W_ao2, b_ao):
    n_atoms, d_atom = atom_features.shape
    d_agg = W_pa_s.shape[0]
    d_out = W_aa.shape[1]

    def body(af_ref, sp_ref, wpaa_ref, wpas_ref, bpa_ref, waa_ref, baa_ref,
             wao1_ref, wao2_ref, bao_ref, out_ref):
        af = af_ref[...]
        s = sp_ref[0:n_atoms, 0:d_agg]
        a_pa = _leaky(
            jnp.dot(af, wpaa_ref[...], preferred_element_type=jnp.float32)
            + jnp.dot(s, wpas_ref[...], preferred_element_type=jnp.float32)
            + bpa_ref[...])
        a_aa = _leaky(
            jnp.dot(af, waa_ref[...], preferred_element_type=jnp.float32)
            + baa_ref[...])
        out_ref[...] = _leaky(
            jnp.dot(a_pa, wao1_ref[...], preferred_element_type=jnp.float32)
            + jnp.dot(a_aa, wao2_ref[...], preferred_element_type=jnp.float32)
            + bao_ref[...])

    return pl.pallas_call(
        body,
        out_shape=jax.ShapeDtypeStruct((n_atoms, d_out), jnp.float32),
    )(atom_features, s_part, W_pa_a, W_pa_s, b_pa, W_aa, b_aa,
      W_ao1, W_ao2, b_ao)


def _tc_pair(pair_features, vv, W_ap_p, b_ap, W_pp, b_pp, W_po1, W_po2, b_po,
             pair_block):
    n_pairs, d_pair = pair_features.shape
    d_out = W_pp.shape[1]
    grid = n_pairs // pair_block

    def body(pf_ref, vv_ref, wapp_ref, bap_ref, wpp_ref, bpp_ref,
             wpo1_ref, wpo2_ref, bpo_ref, out_ref):
        pf = pf_ref[...]
        p_apa = _leaky(
            jnp.dot(pf, wapp_ref[...], preferred_element_type=jnp.float32)
            + bap_ref[...] + vv_ref[...])
        p_pp = _leaky(
            jnp.dot(pf, wpp_ref[...], preferred_element_type=jnp.float32)
            + bpp_ref[...])
        out_ref[...] = _leaky(
            jnp.dot(p_apa, wpo1_ref[...], preferred_element_type=jnp.float32)
            + jnp.dot(p_pp, wpo2_ref[...], preferred_element_type=jnp.float32)
            + bpo_ref[...])

    return pl.pallas_call(
        body,
        grid=(grid,),
        in_specs=[
            pl.BlockSpec((pair_block, d_pair), lambda i: (i, 0)),
            pl.BlockSpec((pair_block, d_out), lambda i: (i, 0)),
            pl.BlockSpec(W_ap_p.shape, lambda i: (0, 0)),
            pl.BlockSpec(b_ap.shape, lambda i: (0, 0)),
            pl.BlockSpec(W_pp.shape, lambda i: (0, 0)),
            pl.BlockSpec(b_pp.shape, lambda i: (0, 0)),
            pl.BlockSpec(W_po1.shape, lambda i: (0, 0)),
            pl.BlockSpec(W_po2.shape, lambda i: (0, 0)),
            pl.BlockSpec(b_po.shape, lambda i: (0, 0)),
        ],
        out_specs=pl.BlockSpec((pair_block, d_out), lambda i: (i, 0)),
        out_shape=jax.ShapeDtypeStruct((n_pairs, d_out), jnp.float32),
    )(pair_features, vv, W_ap_p, b_ap, W_pp, b_pp, W_po1, W_po2, b_po)


def kernel(atom_features, pair_features, pair_split, atom_to_pair, num_atoms,
           W_pap, b_pap, W_pa, b_pa, W_aa, b_aa, W_ao, b_ao,
           W_ap, b_ap, W_pp, b_pp, W_po, b_po):
    del pair_split, num_atoms
    n_atoms, d_atom = atom_features.shape
    n_pairs, d_pair = pair_features.shape
    d_agg = W_pap.shape[1]
    d_out_a = W_pa.shape[1]
    d_out_p = W_pp.shape[1]

    pair_i = atom_to_pair[:, 0]
    pair_j = atom_to_pair[:, 1]

    W_pap_i = W_pap[:d_atom]
    W_pap_p = W_pap[d_atom:d_atom + d_pair]
    W_pap_j = W_pap[d_atom + d_pair:]
    W_pa_a = W_pa[:d_atom]
    W_pa_s = W_pa[d_atom:]
    W_ao1 = W_ao[:d_out_a]
    W_ao2 = W_ao[d_out_a:]
    W_ap_p = W_ap[:d_pair]
    W_ap_a = W_ap[d_pair:]
    W_po1 = W_po[:d_out_p]
    W_po2 = W_po[d_out_p:]

    W_cat = jnp.concatenate([W_pap_i, W_pap_j, W_ap_a], axis=1)

    pair_block = 4000

    tp, t = _tc_pre(atom_features, pair_features, W_cat, W_pap_p,
                    b_pap.reshape(1, -1), pair_block)

    s_part, vv = _sc_gather_scatter(pair_i, pair_j, t, tp, d_agg, d_out_p)

    atom_hidden = _tc_atom(atom_features, s_part, W_pa_a, W_pa_s,
                           b_pa.reshape(1, -1), W_aa, b_aa.reshape(1, -1),
                           W_ao1, W_ao2, b_ao.reshape(1, -1))

    pair_hidden = _tc_pair(pair_features, vv, W_ap_p, b_ap.reshape(1, -1),
                           W_pp, b_pp.reshape(1, -1), W_po1, W_po2,
                           b_po.reshape(1, -1), pair_block)

    return (atom_hidden, pair_hidden)
